# baseline (speedup 1.0000x reference)
import jax
import jax.numpy as jnp
from jax.experimental import pallas as pl


def _linear_relu_kernel(x_ref, wt_ref, b_ref, o_ref):
    acc = jnp.dot(x_ref[...], wt_ref[...], preferred_element_type=jnp.float32)
    o_ref[...] = jnp.maximum(acc + b_ref[...], 0.0)


def kernel(x, W1, b1):
    M, K = x.shape
    N = W1.shape[0]
    BM = 4096
    wt = W1.T
    b = b1[None, :]
    h = pl.pallas_call(
        _linear_relu_kernel,
        grid=(M // BM,),
        in_specs=[
            pl.BlockSpec((BM, K), lambda i: (i, 0)),
            pl.BlockSpec((K, N), lambda i: (0, 0)),
            pl.BlockSpec((1, N), lambda i: (0, 0)),
        ],
        out_specs=pl.BlockSpec((BM, N), lambda i: (i, 0)),
        out_shape=jax.ShapeDtypeStruct((M, N), jnp.float32),
    )(x, wt, b)
    zero = jnp.zeros((), dtype=jnp.float32)
    return (h, zero, zero, zero)


# speedup vs baseline: 1.0018x; 1.0018x over previous
import jax
import jax.numpy as jnp
from jax.experimental import pallas as pl


def _linear_relu_kernel(x_ref, wt_ref, b_ref, o_ref):
    acc = jnp.dot(x_ref[...], wt_ref[...],
                  preferred_element_type=jnp.float32,
                  precision=jax.lax.Precision.DEFAULT)
    o_ref[...] = jnp.maximum(acc + b_ref[...], 0.0)


def kernel(x, W1, b1):
    M, K = x.shape
    N = W1.shape[0]
    BM = 4096
    wt = W1.T
    b = b1[None, :]
    h = pl.pallas_call(
        _linear_relu_kernel,
        grid=(M // BM,),
        in_specs=[
            pl.BlockSpec((BM, K), lambda i: (i, 0)),
            pl.BlockSpec((K, N), lambda i: (0, 0)),
            pl.BlockSpec((1, N), lambda i: (0, 0)),
        ],
        out_specs=pl.BlockSpec((BM, N), lambda i: (i, 0)),
        out_shape=jax.ShapeDtypeStruct((M, N), jnp.float32),
    )(x, wt, b)
    zero = jnp.zeros((), dtype=jnp.float32)
    return (h, zero, zero, zero)


# speedup vs baseline: 1.0485x; 1.0467x over previous
import jax
import jax.numpy as jnp
from jax.experimental import pallas as pl
from jax.experimental.pallas import tpu as pltpu

_M = 65536
_K = 100
_N = 100
_CHUNK = 2048
_NCHUNKS = _M // _CHUNK
_NBUF = 8


def _pipelined_kernel(x_hbm, wt_vmem, b_vmem, o_hbm,
                      xbuf, obuf, in_sems, out_sems):
    def start_in(i):
        slot = i % _NBUF
        pltpu.make_async_copy(
            x_hbm.at[pl.ds(i * _CHUNK, _CHUNK), :],
            xbuf.at[slot],
            in_sems.at[slot],
        ).start()

    def wait_in(i):
        slot = i % _NBUF
        pltpu.make_async_copy(
            x_hbm.at[pl.ds(i * _CHUNK, _CHUNK), :],
            xbuf.at[slot],
            in_sems.at[slot],
        ).wait()

    def start_out(i):
        slot = i % _NBUF
        pltpu.make_async_copy(
            obuf.at[slot],
            o_hbm.at[pl.ds(i * _CHUNK, _CHUNK), :],
            out_sems.at[slot],
        ).start()

    def wait_out(i):
        slot = i % _NBUF
        pltpu.make_async_copy(
            obuf.at[slot],
            o_hbm.at[pl.ds(i * _CHUNK, _CHUNK), :],
            out_sems.at[slot],
        ).wait()

    for i in range(_NBUF):
        start_in(i)

    for i in range(_NCHUNKS):
        wait_in(i)
        if i >= _NBUF:
            wait_out(i - _NBUF)
        acc = jnp.dot(xbuf[i % _NBUF], wt_vmem[...],
                      preferred_element_type=jnp.float32)
        obuf[i % _NBUF] = jnp.maximum(acc + b_vmem[...], 0.0)
        start_out(i)
        if i + _NBUF < _NCHUNKS:
            start_in(i + _NBUF)

    for i in range(_NCHUNKS - _NBUF, _NCHUNKS):
        wait_out(i)


def kernel(x, W1, b1):
    wt = W1.T
    b = b1[None, :]
    h = pl.pallas_call(
        _pipelined_kernel,
        in_specs=[
            pl.BlockSpec(memory_space=pl.ANY),
            pl.BlockSpec(memory_space=pltpu.MemorySpace.VMEM),
            pl.BlockSpec(memory_space=pltpu.MemorySpace.VMEM),
        ],
        out_specs=pl.BlockSpec(memory_space=pl.ANY),
        out_shape=jax.ShapeDtypeStruct((_M, _N), jnp.float32),
        scratch_shapes=[
            pltpu.VMEM((_NBUF, _CHUNK, _K), jnp.float32),
            pltpu.VMEM((_NBUF, _CHUNK, _N), jnp.float32),
            pltpu.SemaphoreType.DMA((_NBUF,)),
            pltpu.SemaphoreType.DMA((_NBUF,)),
        ],
    )(x, wt, b)
    zero = jnp.zeros((), dtype=jnp.float32)
    return (h, zero, zero, zero)


# speedup vs baseline: 3.0626x; 2.9209x over previous
import jax
import jax.numpy as jnp
from jax.experimental import pallas as pl
from jax.experimental.pallas import tpu as pltpu


def _linear_relu_t_kernel(xt_ref, w_ref, b_ref, o_ref):
    acc = jnp.dot(w_ref[...], xt_ref[...], preferred_element_type=jnp.float32)
    o_ref[...] = jnp.maximum(acc + b_ref[...], 0.0)


def kernel(x, W1, b1):
    M, K = x.shape
    N = W1.shape[0]
    BN = 4096
    xt = x.T
    b = b1[:, None]
    ht = pl.pallas_call(
        _linear_relu_t_kernel,
        grid=(M // BN,),
        in_specs=[
            pl.BlockSpec((K, BN), lambda i: (0, i)),
            pl.BlockSpec((N, K), lambda i: (0, 0)),
            pl.BlockSpec((N, 1), lambda i: (0, 0)),
        ],
        out_specs=pl.BlockSpec((N, BN), lambda i: (0, i)),
        out_shape=jax.ShapeDtypeStruct((N, M), jnp.float32),
    )(xt, W1, b)
    h = ht.T
    zero = jnp.zeros((), dtype=jnp.float32)
    return (h, zero, zero, zero)


# speedup vs baseline: 3.5072x; 1.1452x over previous
import jax
import jax.numpy as jnp
from jax.experimental import pallas as pl
from jax.experimental.pallas import tpu as pltpu


def _linear_relu_t_kernel(xt_ref, w_ref, b_ref, o_ref):
    acc = jnp.dot(w_ref[...], xt_ref[...], preferred_element_type=jnp.float32)
    o_ref[...] = jnp.maximum(acc + b_ref[...], 0.0)


def kernel(x, W1, b1):
    M, K = x.shape
    N = W1.shape[0]
    BN = 8192
    xt = x.T
    b = b1[:, None]
    ht = pl.pallas_call(
        _linear_relu_t_kernel,
        grid=(M // BN,),
        in_specs=[
            pl.BlockSpec((K, BN), lambda i: (0, i)),
            pl.BlockSpec((N, K), lambda i: (0, 0)),
            pl.BlockSpec((N, 1), lambda i: (0, 0)),
        ],
        out_specs=pl.BlockSpec((N, BN), lambda i: (0, i)),
        out_shape=jax.ShapeDtypeStruct((N, M), jnp.float32),
    )(xt, W1, b)
    h = ht.T
    zero = jnp.zeros((), dtype=jnp.float32)
    return (h, zero, zero, zero)


# speedup vs baseline: 3.6194x; 1.0320x over previous
import jax
import jax.numpy as jnp
from jax.experimental import pallas as pl
from jax.experimental.pallas import tpu as pltpu


def _linear_relu_t_kernel(xt_ref, w_ref, b_ref, o_ref):
    acc = jnp.dot(w_ref[...], xt_ref[...], preferred_element_type=jnp.float32)
    o_ref[...] = jnp.maximum(acc + b_ref[...], 0.0)


def kernel(x, W1, b1):
    M, K = x.shape
    N = W1.shape[0]
    BN = 16384
    xt = x.T
    b = b1[:, None]
    ht = pl.pallas_call(
        _linear_relu_t_kernel,
        grid=(M // BN,),
        in_specs=[
            pl.BlockSpec((K, BN), lambda i: (0, i)),
            pl.BlockSpec((N, K), lambda i: (0, 0)),
            pl.BlockSpec((N, 1), lambda i: (0, 0)),
        ],
        out_specs=pl.BlockSpec((N, BN), lambda i: (0, i)),
        out_shape=jax.ShapeDtypeStruct((N, M), jnp.float32),
    )(xt, W1, b)
    h = ht.T
    zero = jnp.zeros((), dtype=jnp.float32)
    return (h, zero, zero, zero)


# speedup vs baseline: 3.7506x; 1.0362x over previous
import jax
import jax.numpy as jnp
from jax.experimental import pallas as pl
from jax.experimental.pallas import tpu as pltpu

_M = 65536
_K = 100
_N = 100
_CHUNK = 4096
_NCHUNKS = _M // _CHUNK
_NBUF = 6


def _pipelined_kernel(xt_hbm, w_vmem, b_vmem, o_hbm,
                      xbuf, obuf, in_sems, out_sems):
    def in_copy(i):
        slot = i % _NBUF
        return pltpu.make_async_copy(
            xt_hbm.at[:, pl.ds(i * _CHUNK, _CHUNK)],
            xbuf.at[slot],
            in_sems.at[slot],
        )

    def out_copy(i):
        slot = i % _NBUF
        return pltpu.make_async_copy(
            obuf.at[slot],
            o_hbm.at[:, pl.ds(i * _CHUNK, _CHUNK)],
            out_sems.at[slot],
        )

    for i in range(_NBUF):
        in_copy(i).start()

    for i in range(_NCHUNKS):
        slot = i % _NBUF
        in_copy(i).wait()
        if i >= _NBUF:
            out_copy(i - _NBUF).wait()
        acc = jnp.dot(w_vmem[...], xbuf[slot],
                      preferred_element_type=jnp.float32)
        obuf[slot] = jnp.maximum(acc + b_vmem[...], 0.0)
        out_copy(i).start()
        if i + _NBUF < _NCHUNKS:
            in_copy(i + _NBUF).start()

    for i in range(_NCHUNKS - _NBUF, _NCHUNKS):
        out_copy(i).wait()


def kernel(x, W1, b1):
    xt = x.T
    b = b1[:, None]
    ht = pl.pallas_call(
        _pipelined_kernel,
        in_specs=[
            pl.BlockSpec(memory_space=pl.ANY),
            pl.BlockSpec(memory_space=pltpu.MemorySpace.VMEM),
            pl.BlockSpec(memory_space=pltpu.MemorySpace.VMEM),
        ],
        out_specs=pl.BlockSpec(memory_space=pl.ANY),
        out_shape=jax.ShapeDtypeStruct((_N, _M), jnp.float32),
        scratch_shapes=[
            pltpu.VMEM((_NBUF, _K, _CHUNK), jnp.float32),
            pltpu.VMEM((_NBUF, _N, _CHUNK), jnp.float32),
            pltpu.SemaphoreType.DMA((_NBUF,)),
            pltpu.SemaphoreType.DMA((_NBUF,)),
        ],
    )(xt, W1, b)
    h = ht.T
    zero = jnp.zeros((), dtype=jnp.float32)
    return (h, zero, zero, zero)


# speedup vs baseline: 3.7686x; 1.0048x over previous
import jax
import jax.numpy as jnp
from jax.experimental import pallas as pl
from jax.experimental.pallas import tpu as pltpu

_M = 65536
_K = 100
_N = 100
_CHUNK = 8192
_NCHUNKS = _M // _CHUNK
_NBUF = 4


def _pipelined_kernel(xt_hbm, w_vmem, b_vmem, o_hbm,
                      xbuf, obuf, in_sems, out_sems):
    def in_copy(i):
        slot = i % _NBUF
        return pltpu.make_async_copy(
            xt_hbm.at[:, pl.ds(i * _CHUNK, _CHUNK)],
            xbuf.at[slot],
            in_sems.at[slot],
        )

    def out_copy(i):
        slot = i % _NBUF
        return pltpu.make_async_copy(
            obuf.at[slot],
            o_hbm.at[:, pl.ds(i * _CHUNK, _CHUNK)],
            out_sems.at[slot],
        )

    for i in range(_NBUF):
        in_copy(i).start()

    for i in range(_NCHUNKS):
        slot = i % _NBUF
        in_copy(i).wait()
        if i >= _NBUF:
            out_copy(i - _NBUF).wait()
        acc = jnp.dot(w_vmem[...], xbuf[slot],
                      preferred_element_type=jnp.float32)
        obuf[slot] = jnp.maximum(acc + b_vmem[...], 0.0)
        out_copy(i).start()
        if i + _NBUF < _NCHUNKS:
            in_copy(i + _NBUF).start()

    for i in range(_NCHUNKS - _NBUF, _NCHUNKS):
        out_copy(i).wait()


def kernel(x, W1, b1):
    xt = x.T
    b = b1[:, None]
    ht = pl.pallas_call(
        _pipelined_kernel,
        in_specs=[
            pl.BlockSpec(memory_space=pl.ANY),
            pl.BlockSpec(memory_space=pltpu.MemorySpace.VMEM),
            pl.BlockSpec(memory_space=pltpu.MemorySpace.VMEM),
        ],
        out_specs=pl.BlockSpec(memory_space=pl.ANY),
        out_shape=jax.ShapeDtypeStruct((_N, _M), jnp.float32),
        scratch_shapes=[
            pltpu.VMEM((_NBUF, _K, _CHUNK), jnp.float32),
            pltpu.VMEM((_NBUF, _N, _CHUNK), jnp.float32),
            pltpu.SemaphoreType.DMA((_NBUF,)),
            pltpu.SemaphoreType.DMA((_NBUF,)),
        ],
    )(xt, W1, b)
    h = ht.T
    zero = jnp.zeros((), dtype=jnp.float32)
    return (h, zero, zero, zero)
